# baseline (speedup 1.0000x reference)
import functools

import jax
import jax.numpy as jnp
from jax import lax
from jax.experimental import pallas as pl
from jax.experimental.pallas import tpu as pltpu
from jax.experimental.pallas import tpu_sc as plsc

U = 8192
D_IN = 128
GX = 256
GY = 256
F = 32
GP = 258
XSTR = 264
FP = 272
NFLAT = GP * XSTR
GBIG = 68768
HALF_ROWS = GBIG // 2
TRASH = HALF_ROWS
SP_ROWS = HALF_ROWS + 1
TILE_U = U // 16
ZCHUNK = 2144
ZTAIL = HALF_ROWS - 16 * ZCHUNK
EPS = 1e-6

import numpy as _np
_pf = _np.arange(GBIG) - FP
_MASK_NP = ((_pf >= 0) & (_pf < NFLAT)
            & (_pf % XSTR >= 1) & (_pf % XSTR <= GX)
            & (_pf // XSTR >= 1) & (_pf // XSTR <= GY)).astype(_np.float32)
_MASK_NP = _MASK_NP.reshape(GBIG, 1)


def _ln(x, scale, bias):
    m = jnp.mean(x, axis=-1, keepdims=True)
    v = jnp.mean((x - m) * (x - m), axis=-1, keepdims=True)
    return (x - m) * jax.lax.rsqrt(v + EPS) * scale + bias


def _mlp_body(z_ref, ne_ref, s0_ref, t0_ref, w0_ref, b0_ref,
              s1_ref, t1_ref, w1_ref, b1_ref,
              s2_ref, t2_ref, w2_ref, b2_ref, out_ref):
    h = z_ref[...]
    h = _ln(h, s0_ref[...], t0_ref[...])
    h = jnp.maximum(h, 0.0)
    h = jnp.dot(h, w0_ref[...], preferred_element_type=jnp.float32) + b0_ref[...]
    h = _ln(h, s1_ref[...], t1_ref[...])
    h = jnp.maximum(h, 0.0)
    h = jnp.dot(h, w1_ref[...], preferred_element_type=jnp.float32) + b1_ref[...]
    h = _ln(h, s2_ref[...], t2_ref[...])
    h = jnp.maximum(h, 0.0)
    h = jnp.dot(h, w2_ref[...], preferred_element_type=jnp.float32) + b2_ref[...]
    out_ref[...] = h * ne_ref[...]


def _run_mlp(z, ne_f32, params):
    blk = 512
    grid = U // blk
    full = lambda shape: pl.BlockSpec(shape, lambda i: (0, 0))
    in_specs = [
        pl.BlockSpec((blk, D_IN), lambda i: (i, 0)),
        pl.BlockSpec((blk, 1), lambda i: (i, 0)),
        full((1, 128)), full((1, 128)), full((128, 128)), full((1, 128)),
        full((1, 128)), full((1, 128)), full((128, 64)), full((1, 64)),
        full((1, 64)), full((1, 64)), full((64, 32)), full((1, 32)),
    ]
    return pl.pallas_call(
        _mlp_body,
        grid=(grid,),
        in_specs=in_specs,
        out_specs=pl.BlockSpec((blk, F), lambda i: (i, 0)),
        out_shape=jax.ShapeDtypeStruct((U, F), jnp.float32),
    )(z, ne_f32, *params)


def _scatter_body(h_hbm, x_hbm, y_hbm, zeros_hbm, out_hbm,
                  shared, idx2d, xb, yb, emb):
    c = lax.axis_index("c")
    s = lax.axis_index("s")

    pltpu.sync_copy(zeros_hbm.at[pl.ds(0, ZCHUNK)],
                    shared.at[pl.ds(s * ZCHUNK, ZCHUNK)])

    @pl.when(s == 0)
    def _():
        pltpu.sync_copy(zeros_hbm.at[pl.ds(0, ZTAIL)],
                        shared.at[pl.ds(16 * ZCHUNK, ZTAIL)])

    base_u = s * TILE_U
    pltpu.sync_copy(h_hbm.at[pl.ds(base_u, TILE_U)], emb)
    pltpu.sync_copy(x_hbm.at[pl.ds(base_u, TILE_U)], xb)
    pltpu.sync_copy(y_hbm.at[pl.ds(base_u, TILE_U)], yb)

    half_base = c * HALF_ROWS
    for j in range(TILE_U // 16):
        xv = xb[pl.ds(j * 16, 16)]
        yv = yb[pl.ds(j * 16, 16)]
        flat = yv * XSTR + xv + (FP + XSTR + 1) - half_base
        valid = (flat >= 0) & (flat < HALF_ROWS)
        idx = jnp.where(valid, flat, TRASH)
        idx2d[j // 8, pl.ds((j % 8) * 16, 16)] = idx

    plsc.subcore_barrier()
    for q in range(TILE_U // 128):
        pltpu.sync_copy(emb.at[pl.ds(q * 128, 128)],
                        shared.at[idx2d.at[q]], add=True)
    plsc.subcore_barrier()

    out_base = c * HALF_ROWS + s * ZCHUNK
    pltpu.sync_copy(shared.at[pl.ds(s * ZCHUNK, ZCHUNK)],
                    out_hbm.at[pl.ds(out_base, ZCHUNK)])

    @pl.when(s == 0)
    def _():
        pltpu.sync_copy(shared.at[pl.ds(16 * ZCHUNK, ZTAIL)],
                        out_hbm.at[pl.ds(c * HALF_ROWS + 16 * ZCHUNK, ZTAIL)])


def _run_scatter(h, unit_x, unit_y):
    zeros = jnp.zeros((ZCHUNK, F), jnp.float32)
    mesh = plsc.VectorSubcoreMesh(core_axis_name="c", subcore_axis_name="s",
                                  num_cores=2, num_subcores=16)
    fn = pl.kernel(
        _scatter_body,
        out_type=jax.ShapeDtypeStruct((2 * HALF_ROWS, F), jnp.float32),
        mesh=mesh,
        scratch_types=[
            pltpu.VMEM_SHARED((SP_ROWS, F), jnp.float32),
            pltpu.VMEM((4, 128), jnp.int32),
            pltpu.VMEM((TILE_U,), jnp.int32),
            pltpu.VMEM((TILE_U,), jnp.int32),
            pltpu.VMEM((TILE_U, F), jnp.float32),
        ],
        compiler_params=pltpu.CompilerParams(use_tc_tiling_on_sc=False),
    )
    return fn(h, unit_x, unit_y, zeros)


R_CONV = 4264
SLAB = R_CONV + 544
LEN2 = R_CONV + 528


def _conv_body(g_hbm, m_hbm, sf_ref, tf_ref, wf_ref, cb_ref, out_ref,
               gbuf, mbuf, sem_g, sem_m):
    i = pl.program_id(0)
    slot = lax.rem(i, 2)
    nxt = lax.rem(i + 1, 2)

    def _start(step, buf_slot):
        pltpu.make_async_copy(g_hbm.at[pl.ds(step * R_CONV, SLAB)],
                              gbuf.at[buf_slot], sem_g.at[buf_slot]).start()
        pltpu.make_async_copy(m_hbm.at[pl.ds(step * R_CONV, SLAB)],
                              mbuf.at[buf_slot], sem_m.at[buf_slot]).start()

    @pl.when(i == 0)
    def _():
        _start(i, slot)

    @pl.when(i + 1 < 16)
    def _():
        _start(i + 1, nxt)

    pltpu.make_async_copy(g_hbm.at[pl.ds(i * R_CONV, SLAB)],
                          gbuf.at[slot], sem_g.at[slot]).wait()
    pltpu.make_async_copy(m_hbm.at[pl.ds(i * R_CONV, SLAB)],
                          mbuf.at[slot], sem_m.at[slot]).wait()
    a = _ln(gbuf[slot], sf_ref[...], tf_ref[...])
    a = jnp.maximum(a, 0.0)
    a = a * mbuf[slot]
    acc = jnp.broadcast_to(cb_ref[...], (R_CONV, F))
    for kw in range(3):
        akw = lax.slice(a, (7 + kw, 0), (7 + kw + LEN2, F))
        p = jnp.dot(akw, wf_ref[kw], preferred_element_type=jnp.float32)
        for kh in range(3):
            acc = acc + lax.slice(p, (264 * kh, F * kh),
                                  (264 * kh + R_CONV, F * kh + F))
    out_ref[...] = acc


def _run_conv(grid_big, ln_scale_f, ln_bias_f, conv_w, conv_b):
    wf = conv_w.transpose(1, 2, 0, 3).reshape(3, F, 3 * F)
    mask = jnp.asarray(_MASK_NP)
    out = pl.pallas_call(
        _conv_body,
        grid=(16,),
        in_specs=[
            pl.BlockSpec(memory_space=pl.ANY),
            pl.BlockSpec(memory_space=pl.ANY),
            pl.BlockSpec((1, F), lambda i: (0, 0)),
            pl.BlockSpec((1, F), lambda i: (0, 0)),
            pl.BlockSpec((3, F, 3 * F), lambda i: (0, 0, 0)),
            pl.BlockSpec((1, F), lambda i: (0, 0)),
        ],
        out_specs=pl.BlockSpec((R_CONV, F), lambda i: (i, 0)),
        out_shape=jax.ShapeDtypeStruct((16 * R_CONV, F), jnp.float32),
        scratch_shapes=[
            pltpu.VMEM((2, SLAB, F), jnp.float32),
            pltpu.VMEM((2, SLAB, 1), jnp.float32),
            pltpu.SemaphoreType.DMA((2,)),
            pltpu.SemaphoreType.DMA((2,)),
        ],
        compiler_params=pltpu.CompilerParams(vmem_limit_bytes=63 * 2**20),
    )(grid_big, mask, ln_scale_f.reshape(1, F), ln_bias_f.reshape(1, F),
      wf, conv_b.reshape(1, F))
    return out[:NFLAT].reshape(GP, XSTR, F)[1:GY + 1, 1:GX + 1, :]


def kernel(z, unit_x, unit_y, non_empty_units, ln_scale_0, ln_bias_0, w0, b0,
           ln_scale_1, ln_bias_1, w1, b1, ln_scale_2, ln_bias_2, w2, b2,
           ln_scale_f, ln_bias_f, conv_w, conv_b):
    ne = non_empty_units.astype(jnp.float32).reshape(U, 1)
    params = (ln_scale_0.reshape(1, -1), ln_bias_0.reshape(1, -1), w0,
              b0.reshape(1, -1),
              ln_scale_1.reshape(1, -1), ln_bias_1.reshape(1, -1), w1,
              b1.reshape(1, -1),
              ln_scale_2.reshape(1, -1), ln_bias_2.reshape(1, -1), w2,
              b2.reshape(1, -1))
    h = _run_mlp(z, ne, params)
    grid_big = _run_scatter(h, unit_x, unit_y)
    return _run_conv(grid_big, ln_scale_f, ln_bias_f, conv_w, conv_b)


# speedup vs baseline: 1.8093x; 1.8093x over previous
import functools

import jax
import jax.numpy as jnp
import numpy as _np
from jax import lax
from jax.experimental import pallas as pl
from jax.experimental.pallas import tpu as pltpu
from jax.experimental.pallas import tpu_sc as plsc

U = 8192
D_IN = 128
GX = 256
GY = 256
F = 32
PPR = 4
LW = 128
XSTR = 288
GP = 258
NFLAT = GP * XSTR
NP = NFLAT // PPR
FP4 = 80
R4 = 2448
NSTEP = 8
SLAB4 = 2624
PBIG = (NSTEP - 1) * R4 + SLAB4
HALF4 = PBIG // 2
TRASH = HALF4
SPR = HALF4 + 1
TILE_U = U // 16
ZCH = 616
ZTL = HALF4 - 16 * ZCH
ROW_LO = 152
ROW_HI = 18584
EPS = 1e-6

_l = _np.arange(SLAB4)
_XMASK_NP = (((_l - FP4) % 72) < 64).astype(_np.float32).reshape(SLAB4, 1)


def _ln(x, scale, bias):
    m = jnp.mean(x, axis=-1, keepdims=True)
    v = jnp.mean((x - m) * (x - m), axis=-1, keepdims=True)
    return (x - m) * jax.lax.rsqrt(v + EPS) * scale + bias


def _mlp_body(z_ref, ne_ref, xq_ref, t4_ref, s0_ref, t0_ref, w0_ref, b0_ref,
              s1_ref, t1_ref, w1_ref, b1_ref,
              s2_ref, t2_ref, w2_ref, b2_ref, out_ref):
    h = z_ref[...]
    h = _ln(h, s0_ref[...], t0_ref[...])
    h = jnp.maximum(h, 0.0)
    h = jnp.dot(h, w0_ref[...], preferred_element_type=jnp.float32) + b0_ref[...]
    h = _ln(h, s1_ref[...], t1_ref[...])
    h = jnp.maximum(h, 0.0)
    h = jnp.dot(h, w1_ref[...], preferred_element_type=jnp.float32) + b1_ref[...]
    h = _ln(h, s2_ref[...], t2_ref[...])
    h = jnp.maximum(h, 0.0)
    h = jnp.dot(h, w2_ref[...], preferred_element_type=jnp.float32) + b2_ref[...]
    h = h * ne_ref[...]
    laneq = lax.broadcasted_iota(jnp.int32, (1, LW), 1) // F
    m4 = (laneq == xq_ref[...]).astype(jnp.float32)
    out_ref[...] = jnp.dot(h, t4_ref[...],
                           preferred_element_type=jnp.float32) * m4


def _run_mlp(z, ne_f32, xq, params):
    blk = 1024
    grid = U // blk
    t4 = jnp.tile(jnp.eye(F, dtype=jnp.float32), (1, PPR))
    full = lambda shape: pl.BlockSpec(shape, lambda i: (0, 0))
    in_specs = [
        pl.BlockSpec((blk, D_IN), lambda i: (i, 0)),
        pl.BlockSpec((blk, 1), lambda i: (i, 0)),
        pl.BlockSpec((blk, 1), lambda i: (i, 0)),
        full((F, LW)),
        full((1, 128)), full((1, 128)), full((128, 128)), full((1, 128)),
        full((1, 128)), full((1, 128)), full((128, 64)), full((1, 64)),
        full((1, 64)), full((1, 64)), full((64, 32)), full((1, 32)),
    ]
    return pl.pallas_call(
        _mlp_body,
        grid=(grid,),
        in_specs=in_specs,
        out_specs=pl.BlockSpec((blk, LW), lambda i: (i, 0)),
        out_shape=jax.ShapeDtypeStruct((U, LW), jnp.float32),
    )(z, ne_f32, xq, t4, *params)


def _scatter_body(h_hbm, x_hbm, y_hbm, zeros_hbm, out_hbm,
                  shared, idx2d, xb, yb, emb):
    c = lax.axis_index("c")
    s = lax.axis_index("s")

    pltpu.sync_copy(zeros_hbm.at[pl.ds(0, ZCH)],
                    shared.at[pl.ds(s * ZCH, ZCH)])

    @pl.when(s == 0)
    def _():
        pltpu.sync_copy(zeros_hbm.at[pl.ds(0, ZTL)],
                        shared.at[pl.ds(16 * ZCH, ZTL)])

    base_u = s * TILE_U
    pltpu.sync_copy(x_hbm.at[pl.ds(base_u, TILE_U)], xb)
    pltpu.sync_copy(y_hbm.at[pl.ds(base_u, TILE_U)], yb)

    half_base = c * HALF4
    plsc.subcore_barrier()
    for r in range(2):
        pltpu.sync_copy(h_hbm.at[pl.ds(base_u + r * 256, 256)], emb)
        for j in range(16):
            xv = xb[pl.ds(r * 256 + j * 16, 16)]
            yv = yb[pl.ds(r * 256 + j * 16, 16)]
            pf = yv * XSTR + xv + XSTR
            row = lax.shift_right_logical(pf, 2) + (FP4 - half_base)
            valid = (row >= 0) & (row < HALF4)
            idx = jnp.where(valid, row, TRASH)
            idx2d[j // 8, pl.ds((j % 8) * 16, 16)] = idx
        for q in range(2):
            pltpu.sync_copy(emb.at[pl.ds(q * 128, 128)],
                            shared.at[idx2d.at[q]], add=True)
    plsc.subcore_barrier()

    out_base = c * HALF4 + s * ZCH
    pltpu.sync_copy(shared.at[pl.ds(s * ZCH, ZCH)],
                    out_hbm.at[pl.ds(out_base, ZCH)])

    @pl.when(s == 0)
    def _():
        pltpu.sync_copy(shared.at[pl.ds(16 * ZCH, ZTL)],
                        out_hbm.at[pl.ds(c * HALF4 + 16 * ZCH, ZTL)])


def _run_scatter(h128, unit_x, unit_y):
    zeros = jnp.zeros((ZCH, LW), jnp.float32)
    mesh = plsc.VectorSubcoreMesh(core_axis_name="c", subcore_axis_name="s",
                                  num_cores=2, num_subcores=16)
    fn = pl.kernel(
        _scatter_body,
        out_type=jax.ShapeDtypeStruct((PBIG, LW), jnp.float32),
        mesh=mesh,
        scratch_types=[
            pltpu.VMEM_SHARED((SPR, LW), jnp.float32),
            pltpu.VMEM((2, 128), jnp.int32),
            pltpu.VMEM((TILE_U,), jnp.int32),
            pltpu.VMEM((TILE_U,), jnp.int32),
            pltpu.VMEM((256, LW), jnp.float32),
        ],
        compiler_params=pltpu.CompilerParams(use_tc_tiling_on_sc=True),
    )
    return fn(h128, unit_x, unit_y, zeros)


def _conv_body(g_hbm, xm_ref, bd_ref, sf_ref, tf_ref, w9_ref, cb_ref,
               out_ref, gbuf, sem_g):
    i = pl.program_id(0)
    slot = lax.rem(i, 2)
    nxt = lax.rem(i + 1, 2)

    def _start(step, buf_slot):
        pltpu.make_async_copy(g_hbm.at[pl.ds(step * R4, SLAB4)],
                              gbuf.at[buf_slot], sem_g.at[buf_slot]).start()

    @pl.when(i == 0)
    def _():
        _start(i, slot)

    @pl.when(i + 1 < NSTEP)
    def _():
        _start(i + 1, nxt)

    pltpu.make_async_copy(g_hbm.at[pl.ds(i * R4, SLAB4)],
                          gbuf.at[slot], sem_g.at[slot]).wait()

    g = gbuf[slot]
    m1 = jnp.dot(g, bd_ref[...], preferred_element_type=jnp.float32)
    q2 = jnp.dot(g * g, bd_ref[...], preferred_element_type=jnp.float32)
    v = q2 - m1 * m1
    a = (g - m1) * jax.lax.rsqrt(v + EPS) * sf_ref[...] + tf_ref[...]
    a = jnp.maximum(a, 0.0)
    rows = lax.broadcasted_iota(jnp.int32, (SLAB4, 1), 0) + i * R4
    rm = ((rows >= ROW_LO) & (rows < ROW_HI)).astype(jnp.float32)
    a = a * (xm_ref[...] * rm)

    wl = SLAB4 - 16
    a0 = lax.slice(a, (8, 0), (8 + wl, LW))
    anx = lax.slice(a, (9, 0), (9 + wl, LW))
    apv = lax.slice(a, (7, 0), (7 + wl, LW))
    lane = lax.broadcasted_iota(jnp.int32, (wl, LW), 1)
    ap1 = jnp.where(lane < 96, pltpu.roll(a0, LW - F, 1),
                    pltpu.roll(anx, 96, 1))
    am1 = jnp.where(lane >= F, pltpu.roll(a0, F, 1), pltpu.roll(apv, F, 1))
    variants = (am1, a0, ap1)

    acc = jnp.broadcast_to(cb_ref[...], (R4, LW))
    for kh in range(3):
        for kw in range(3):
            sl = lax.slice(variants[kw], (72 * kh, 0), (72 * kh + R4, LW))
            acc = acc + jnp.dot(sl, w9_ref[3 * kh + kw],
                                preferred_element_type=jnp.float32)
    out_ref[...] = acc


def _run_conv(grid_p, ln_scale_f, ln_bias_f, conv_w, conv_b):
    eye4 = jnp.eye(PPR, dtype=jnp.float32)
    w9 = jnp.stack([jnp.kron(eye4, conv_w[kh, kw])
                    for kh in range(3) for kw in range(3)])
    bd = jnp.kron(eye4, jnp.full((F, F), 1.0 / F, jnp.float32))
    tile4 = lambda ve: jnp.tile(ve.reshape(1, F), (1, PPR))
    xmask = jnp.asarray(_XMASK_NP)
    out = pl.pallas_call(
        _conv_body,
        grid=(NSTEP,),
        in_specs=[
            pl.BlockSpec(memory_space=pl.ANY),
            pl.BlockSpec((SLAB4, 1), lambda i: (0, 0)),
            pl.BlockSpec((LW, LW), lambda i: (0, 0)),
            pl.BlockSpec((1, LW), lambda i: (0, 0)),
            pl.BlockSpec((1, LW), lambda i: (0, 0)),
            pl.BlockSpec((9, LW, LW), lambda i: (0, 0, 0)),
            pl.BlockSpec((1, LW), lambda i: (0, 0)),
        ],
        out_specs=pl.BlockSpec((R4, LW), lambda i: (i, 0)),
        out_shape=jax.ShapeDtypeStruct((NSTEP * R4, LW), jnp.float32),
        scratch_shapes=[
            pltpu.VMEM((2, SLAB4, LW), jnp.float32),
            pltpu.SemaphoreType.DMA((2,)),
        ],
        compiler_params=pltpu.CompilerParams(vmem_limit_bytes=63 * 2**20),
    )(grid_p, xmask, bd, tile4(ln_scale_f), tile4(ln_bias_f), w9,
      tile4(conv_b))
    out = out[:GP * 72].reshape(GP, 72, LW)[1:GY + 1, :GX // PPR, :]
    return out.reshape(GY, GX, F)


def kernel(z, unit_x, unit_y, non_empty_units, ln_scale_0, ln_bias_0, w0, b0,
           ln_scale_1, ln_bias_1, w1, b1, ln_scale_2, ln_bias_2, w2, b2,
           ln_scale_f, ln_bias_f, conv_w, conv_b):
    ne = non_empty_units.astype(jnp.float32).reshape(U, 1)
    xq = (unit_x & 3).reshape(U, 1)
    params = (ln_scale_0.reshape(1, -1), ln_bias_0.reshape(1, -1), w0,
              b0.reshape(1, -1),
              ln_scale_1.reshape(1, -1), ln_bias_1.reshape(1, -1), w1,
              b1.reshape(1, -1),
              ln_scale_2.reshape(1, -1), ln_bias_2.reshape(1, -1), w2,
              b2.reshape(1, -1))
    h128 = _run_mlp(z, ne, xq, params)
    grid_p = _run_scatter(h128, unit_x, unit_y)
    return _run_conv(grid_p, ln_scale_f, ln_bias_f, conv_w, conv_b)
